# baseline (speedup 1.0000x reference)
import functools

import jax
import jax.numpy as jnp
from jax import lax
from jax.experimental import pallas as pl
from jax.experimental.pallas import tpu as pltpu
from jax.experimental.pallas import tpu_sc as plsc

_N = 10000
_E = 320000
_D = 128
_H = 128
_EPS = 1e-5

_NC = 2
_NS = 16
_NW = _NC * _NS
_EPW = _E // _NW
_CH = 64
_NITER = _EPW // _CH
_TAIL = _EPW - _NITER * _CH
_NR = 3
_NI = 6

_RSZ = 640
_RLAST = _N - (_NS - 1) * _RSZ

_sc_mesh = plsc.VectorSubcoreMesh(core_axis_name="c", subcore_axis_name="s")


def _rsqrt16(x):
    xi = lax.bitcast_convert_type(x, jnp.int32)
    yi = jnp.int32(0x5F3759DF) - lax.shift_right_arithmetic(xi, 1)
    y = lax.bitcast_convert_type(yi, jnp.float32)
    for _ in range(3):
        y = y * (1.5 - 0.5 * x * y * y)
    return y


_DEG_CH2 = 80
_DEG_EPT = _E // _NS
_DEG_NIT = _DEG_EPT // _DEG_CH2


@functools.partial(
    pl.kernel,
    mesh=_sc_mesh,
    out_type=jax.ShapeDtypeStruct((_N,), jnp.float32),
    scratch_types=[
        pltpu.VMEM((_DEG_NIT, _DEG_CH2), jnp.int32),
        pltpu.VMEM((_DEG_CH2,), jnp.float32),
        pltpu.VMEM((_RSZ,), jnp.float32),
        pltpu.VMEM_SHARED((_N,), jnp.float32),
        pltpu.SemaphoreType.DMA,
    ],
)
def _deg_dinv(dstd_hbm, out_hbm, idx_d, ones, dbuf, acc, sem):
    c = lax.axis_index("c")
    s = lax.axis_index("s")

    @pl.when(c == 0)
    def _core0():
        for j in range(_DEG_CH2 // 16):
            ones[pl.ds(j * 16, 16)] = jnp.ones((16,), jnp.float32)

        def zb(j, carry):
            dbuf[pl.ds(j * 16, 16)] = jnp.zeros((16,), jnp.float32)
            return carry

        lax.fori_loop(0, _RSZ // 16, zb, 0)
        r0 = s * _RSZ

        @pl.when(s < _NS - 1)
        def _():
            pltpu.sync_copy(dbuf.at[pl.ds(0, _RSZ)], acc.at[pl.ds(r0, _RSZ)])

        @pl.when(s == _NS - 1)
        def _():
            pltpu.sync_copy(dbuf.at[pl.ds(0, _RLAST)], acc.at[pl.ds(r0, _RLAST)])

        pltpu.sync_copy(dstd_hbm.at[s], idx_d)
        plsc.subcore_barrier()

        def body(g, carry):
            pltpu.sync_copy(ones, acc.at[idx_d.at[g]], add=True)
            return carry

        lax.fori_loop(0, _DEG_NIT, body, 0)
        plsc.subcore_barrier()

        nvec = jnp.where(s == _NS - 1, _RLAST // 16, _RSZ // 16)

        @pl.when(s < _NS - 1)
        def _():
            pltpu.sync_copy(acc.at[pl.ds(r0, _RSZ)], dbuf.at[pl.ds(0, _RSZ)])

        @pl.when(s == _NS - 1)
        def _():
            pltpu.sync_copy(acc.at[pl.ds(r0, _RLAST)], dbuf.at[pl.ds(0, _RLAST)])

        def nbody(j, carry):
            v = dbuf[pl.ds(j * 16, 16)]
            dbuf[pl.ds(j * 16, 16)] = _rsqrt16(v + 1.0)
            return carry

        lax.fori_loop(0, nvec, nbody, 0)

        @pl.when(s < _NS - 1)
        def _():
            pltpu.sync_copy(dbuf.at[pl.ds(0, _RSZ)], out_hbm.at[pl.ds(r0, _RSZ)])

        @pl.when(s == _NS - 1)
        def _():
            pltpu.sync_copy(dbuf.at[pl.ds(0, _RLAST)], out_hbm.at[pl.ds(r0, _RLAST)])


@functools.partial(
    pl.kernel,
    mesh=_sc_mesh,
    out_type=jax.ShapeDtypeStruct((_NC, _N, _H), jnp.float32),
    scratch_types=[
        pltpu.VMEM((_NI, _CH), jnp.int32),
        pltpu.VMEM((_NI, _CH), jnp.int32),
        pltpu.VMEM((_TAIL,), jnp.int32),
        pltpu.VMEM((_TAIL,), jnp.int32),
        pltpu.VMEM((_NR, _CH, _H), jnp.float32),
        pltpu.VMEM_SHARED((_N, _H), jnp.float32),
        pltpu.SemaphoreType.DMA,
        pltpu.SemaphoreType.DMA,
        pltpu.SemaphoreType.DMA,
        pltpu.SemaphoreType.DMA,
        pltpu.SemaphoreType.DMA,
        pltpu.SemaphoreType.DMA,
        pltpu.SemaphoreType.DMA,
        pltpu.SemaphoreType.DMA,
        pltpu.SemaphoreType.DMA,
        pltpu.SemaphoreType.DMA,
        pltpu.SemaphoreType.DMA,
        pltpu.SemaphoreType.DMA,
    ],
)
def _edge_scatter(xws_hbm, src_hbm, dst_hbm, out_hbm,
                  idx_s, idx_d, tidx_s, tidx_d, rows, acc,
                  is0, is1, is2, is3, is4, is5, gs0, gs1, gs2,
                  ss0, ss1, ss2):
    isems = (is0, is1, is2, is3, is4, is5)
    gsems = (gs0, gs1, gs2)
    ssems = (ss0, ss1, ss2)
    c = lax.axis_index("c")
    s = lax.axis_index("s")
    w = c * _NS + s
    r0 = s * _RSZ
    base0 = w * _EPW

    def zr(i, carry):
        for j in range(_H // 16):
            rows[0, i, pl.ds(j * 16, 16)] = jnp.zeros((16,), jnp.float32)
        return carry

    lax.fori_loop(0, _CH, zr, 0)
    zch = rows.at[0]

    @pl.when(s < _NS - 1)
    def _():
        for k in range(_RSZ // _CH):
            pltpu.sync_copy(zch, acc.at[pl.ds(r0 + k * _CH, _CH)])

    @pl.when(s == _NS - 1)
    def _():
        for k in range(_RLAST // _CH):
            pltpu.sync_copy(zch, acc.at[pl.ds(r0 + k * _CH, _CH)])
        kz = _RLAST // _CH
        pltpu.sync_copy(rows.at[0, pl.ds(0, _RLAST - kz * _CH)],
                        acc.at[pl.ds(r0 + kz * _CH, _RLAST - kz * _CH)])

    plsc.subcore_barrier()

    def issue_idx(j, q6):
        pltpu.async_copy(src_hbm.at[pl.ds(base0 + j * _CH, _CH)],
                         idx_s.at[q6], isems[q6])
        pltpu.async_copy(dst_hbm.at[pl.ds(base0 + j * _CH, _CH)],
                         idx_d.at[q6], isems[q6])

    def wait_idx(q6):
        pltpu.make_async_copy(src_hbm.at[pl.ds(base0, _CH)], idx_s.at[q6],
                              isems[q6]).wait()
        pltpu.make_async_copy(dst_hbm.at[pl.ds(base0, _CH)], idx_d.at[q6],
                              isems[q6]).wait()

    def issue_gather(q6, q3):
        pltpu.async_copy(xws_hbm.at[idx_s.at[q6]], rows.at[q3], gsems[q3])

    def wait_gather(q3):
        pltpu.make_async_copy(xws_hbm.at[idx_s.at[0]], rows.at[q3],
                              gsems[q3]).wait()

    def issue_scat(q6, q3):
        pltpu.async_copy(rows.at[q3], acc.at[idx_d.at[q6]], ssems[q3],
                         add=True)

    def wait_scat(q3):
        pltpu.make_async_copy(rows.at[q3], acc.at[idx_d.at[0]],
                              ssems[q3]).wait()

    for j in range(4):
        issue_idx(j, j)
    wait_idx(0)
    issue_gather(0, 0)
    wait_idx(1)
    issue_gather(1, 1)

    def step(i, u):
        q3 = u % _NR
        q6 = u % _NI
        wait_gather(q3)
        issue_scat(q6, q3)
        j2 = i + 2
        r2 = (u + 2) % _NR
        q62 = (u + 2) % _NI

        @pl.when(j2 < _NITER)
        def _():
            @pl.when(j2 >= _NR)
            def _():
                wait_scat(r2)

            wait_idx(q62)
            issue_gather(q62, r2)

        j4 = i + 4
        q64 = (u + 4) % _NI

        @pl.when(j4 < _NITER)
        def _():
            issue_idx(j4, q64)

    def body(p, carry):
        for u in range(_NI):
            step(p * _NI + u, u)
        return carry

    lax.fori_loop(0, _NITER // _NI, body, 0)
    for q in range(_NR):
        wait_scat(q)

    tb = base0 + _NITER * _CH
    pltpu.sync_copy(src_hbm.at[pl.ds(tb, _TAIL)], tidx_s)
    pltpu.sync_copy(dst_hbm.at[pl.ds(tb, _TAIL)], tidx_d)
    trows = rows.at[0, pl.ds(0, _TAIL)]
    pltpu.async_copy(xws_hbm.at[tidx_s], trows, gs0).wait()
    pltpu.sync_copy(trows, acc.at[tidx_d], add=True)

    plsc.subcore_barrier()

    @pl.when(s < _NS - 1)
    def _():
        pltpu.sync_copy(acc.at[pl.ds(r0, _RSZ)], out_hbm.at[c, pl.ds(r0, _RSZ)])

    @pl.when(s == _NS - 1)
    def _():
        pltpu.sync_copy(acc.at[pl.ds(r0, _RLAST)],
                        out_hbm.at[c, pl.ds(r0, _RLAST)])


_RB = 1000
_G = _N // _RB


def _prep_body(dinv_ref, x_ref, w_ref, xw_ref, xws_ref):
    xw = jnp.dot(x_ref[...], w_ref[...], preferred_element_type=jnp.float32)
    xw_ref[...] = xw
    xws_ref[...] = dinv_ref[...] * xw


_prep = pl.pallas_call(
    _prep_body,
    grid=(_G,),
    in_specs=[
        pl.BlockSpec((_RB, 1), lambda i: (i, 0)),
        pl.BlockSpec((_RB, _D), lambda i: (i, 0)),
        pl.BlockSpec((_D, _H), lambda i: (0, 0)),
    ],
    out_specs=[
        pl.BlockSpec((_RB, _H), lambda i: (i, 0)),
        pl.BlockSpec((_RB, _H), lambda i: (i, 0)),
    ],
    out_shape=[
        jax.ShapeDtypeStruct((_N, _H), jnp.float32),
        jax.ShapeDtypeStruct((_N, _H), jnp.float32),
    ],
)


def _fused1_body(scat_ref, xw_ref, dinv_ref, b_ref, g_ref, be_ref, w2_ref,
                h_ref, xw2_ref, xws2_ref, hbuf, stbuf):
    ph = pl.program_id(0)
    j = pl.program_id(1)

    @pl.when(ph == 0)
    def _():
        dinv = dinv_ref[...]
        scat = scat_ref[0] + scat_ref[1]
        agg = dinv * scat + (dinv * dinv) * xw_ref[...] + b_ref[...]
        h = jnp.maximum(agg, 0.0)
        hbuf[pl.ds(j * _RB, _RB), :] = h

        @pl.when(j == 0)
        def _():
            stbuf[...] = jnp.zeros((2, _H), jnp.float32)

        stbuf[0, :] += jnp.sum(h, axis=0)
        stbuf[1, :] += jnp.sum(h * h, axis=0)

    @pl.when(ph == 1)
    def _():
        mu = stbuf[0, :] * (1.0 / _N)
        var = stbuf[1, :] * (1.0 / _N) - mu * mu
        rs = lax.rsqrt(var + _EPS)
        h = ((hbuf[pl.ds(j * _RB, _RB), :] - mu) * (rs * g_ref[...])
             + be_ref[...])
        h_ref[...] = h
        xw2 = jnp.dot(h, w2_ref[...], preferred_element_type=jnp.float32)
        xw2_ref[...] = xw2
        xws2_ref[...] = dinv_ref[...] * xw2


_fused1 = pl.pallas_call(
    _fused1_body,
    grid=(2, _G),
    in_specs=[
        pl.BlockSpec((_NC, _RB, _H), lambda p, i: (0, i, 0)),
        pl.BlockSpec((_RB, _H), lambda p, i: (i, 0)),
        pl.BlockSpec((_RB, 1), lambda p, i: (i, 0)),
        pl.BlockSpec((1, _H), lambda p, i: (0, 0)),
        pl.BlockSpec((1, _H), lambda p, i: (0, 0)),
        pl.BlockSpec((1, _H), lambda p, i: (0, 0)),
        pl.BlockSpec((_D, _H), lambda p, i: (0, 0)),
    ],
    out_specs=[
        pl.BlockSpec((_RB, _H), lambda p, i: (i, 0)),
        pl.BlockSpec((_RB, _H), lambda p, i: (i, 0)),
        pl.BlockSpec((_RB, _H), lambda p, i: (i, 0)),
    ],
    out_shape=[
        jax.ShapeDtypeStruct((_N, _H), jnp.float32),
        jax.ShapeDtypeStruct((_N, _H), jnp.float32),
        jax.ShapeDtypeStruct((_N, _H), jnp.float32),
    ],
    scratch_shapes=[
        pltpu.VMEM((_N, _H), jnp.float32),
        pltpu.VMEM((2, _H), jnp.float32),
    ],
)


def _fused2_body(scat_ref, xw_ref, dinv_ref, b_ref, g_ref, be_ref,
                 h_ref, hbuf, stbuf):
    ph = pl.program_id(0)
    j = pl.program_id(1)

    @pl.when(ph == 0)
    def _():
        dinv = dinv_ref[...]
        scat = scat_ref[0] + scat_ref[1]
        agg = dinv * scat + (dinv * dinv) * xw_ref[...] + b_ref[...]
        h = jnp.maximum(agg, 0.0)
        hbuf[pl.ds(j * _RB, _RB), :] = h

        @pl.when(j == 0)
        def _():
            stbuf[...] = jnp.zeros((2, _H), jnp.float32)

        stbuf[0, :] += jnp.sum(h, axis=0)
        stbuf[1, :] += jnp.sum(h * h, axis=0)

    @pl.when(ph == 1)
    def _():
        mu = stbuf[0, :] * (1.0 / _N)
        var = stbuf[1, :] * (1.0 / _N) - mu * mu
        rs = lax.rsqrt(var + _EPS)
        h_ref[...] = ((hbuf[pl.ds(j * _RB, _RB), :] - mu) * (rs * g_ref[...])
                      + be_ref[...])


_fused2 = pl.pallas_call(
    _fused2_body,
    grid=(2, _G),
    in_specs=[
        pl.BlockSpec((_NC, _RB, _H), lambda p, i: (0, i, 0)),
        pl.BlockSpec((_RB, _H), lambda p, i: (i, 0)),
        pl.BlockSpec((_RB, 1), lambda p, i: (i, 0)),
        pl.BlockSpec((1, _H), lambda p, i: (0, 0)),
        pl.BlockSpec((1, _H), lambda p, i: (0, 0)),
        pl.BlockSpec((1, _H), lambda p, i: (0, 0)),
    ],
    out_specs=pl.BlockSpec((_RB, _H), lambda p, i: (i, 0)),
    out_shape=jax.ShapeDtypeStruct((_N, _H), jnp.float32),
    scratch_shapes=[
        pltpu.VMEM((_N, _H), jnp.float32),
        pltpu.VMEM((2, _H), jnp.float32),
    ],
)


def kernel(node_features, edge_indices, W1, b1, g1, be1, W2, b2, g2, be2):
    src = edge_indices[0]
    dst = edge_indices[1]
    dst_deg = dst.reshape(_NS, _DEG_NIT, _DEG_CH2)
    dinv = _deg_dinv(dst_deg)
    dinv_col = dinv.reshape(_N, 1)

    xw1, xws1 = _prep(dinv_col, node_features, W1)
    scat1 = _edge_scatter(xws1, src, dst)
    h1, xw2, xws2 = _fused1(scat1, xw1, dinv_col, b1.reshape(1, _H),
                            g1.reshape(1, _H), be1.reshape(1, _H), W2)

    scat2 = _edge_scatter(xws2, src, dst)
    h2 = _fused2(scat2, xw2, dinv_col, b2.reshape(1, _H),
                 g2.reshape(1, _H), be2.reshape(1, _H))
    return (h1, h2)


# speedup vs baseline: 1.0377x; 1.0377x over previous
import functools

import jax
import jax.numpy as jnp
from jax import lax
from jax.experimental import pallas as pl
from jax.experimental.pallas import tpu as pltpu
from jax.experimental.pallas import tpu_sc as plsc

_N = 10000
_E = 320000
_D = 128
_H = 128
_EPS = 1e-5

_NC = 2
_NS = 16
_NW = _NC * _NS
_EPW = _E // _NW
_CH = 64
_NITER = _EPW // _CH
_TAIL = _EPW - _NITER * _CH
_NR = 3
_NI = 6

_RSZ = 640
_RLAST = _N - (_NS - 1) * _RSZ

_sc_mesh = plsc.VectorSubcoreMesh(core_axis_name="c", subcore_axis_name="s")


def _rsqrt16(x):
    xi = lax.bitcast_convert_type(x, jnp.int32)
    yi = jnp.int32(0x5F3759DF) - lax.shift_right_arithmetic(xi, 1)
    y = lax.bitcast_convert_type(yi, jnp.float32)
    for _ in range(3):
        y = y * (1.5 - 0.5 * x * y * y)
    return y


_DEG_CH2 = 80
_DEG_EPT = _E // _NS
_DEG_NIT = _DEG_EPT // _DEG_CH2


@functools.partial(
    pl.kernel,
    mesh=_sc_mesh,
    out_type=jax.ShapeDtypeStruct((_N,), jnp.float32),
    scratch_types=[
        pltpu.VMEM((_DEG_NIT, _DEG_CH2), jnp.int32),
        pltpu.VMEM((_DEG_CH2,), jnp.float32),
        pltpu.VMEM((_RSZ,), jnp.float32),
        pltpu.VMEM_SHARED((_N,), jnp.float32),
        pltpu.SemaphoreType.DMA,
        pltpu.SemaphoreType.DMA,
    ],
)
def _deg_dinv(dstd_hbm, out_hbm, idx_d, ones, dbuf, acc, dg0, dg1):
    c = lax.axis_index("c")
    s = lax.axis_index("s")

    @pl.when(c == 0)
    def _core0():
        for j in range(_DEG_CH2 // 16):
            ones[pl.ds(j * 16, 16)] = jnp.ones((16,), jnp.float32)

        def zb(j, carry):
            dbuf[pl.ds(j * 16, 16)] = jnp.zeros((16,), jnp.float32)
            return carry

        lax.fori_loop(0, _RSZ // 16, zb, 0)
        r0 = s * _RSZ

        @pl.when(s < _NS - 1)
        def _():
            pltpu.sync_copy(dbuf.at[pl.ds(0, _RSZ)], acc.at[pl.ds(r0, _RSZ)])

        @pl.when(s == _NS - 1)
        def _():
            pltpu.sync_copy(dbuf.at[pl.ds(0, _RLAST)], acc.at[pl.ds(r0, _RLAST)])

        pltpu.sync_copy(dstd_hbm.at[s], idx_d)
        plsc.subcore_barrier()

        dsems = (dg0, dg1)

        def body(p, carry):
            for u in range(2):
                g = p * 2 + u

                @pl.when(g >= 2)
                def _():
                    pltpu.make_async_copy(ones, acc.at[idx_d.at[0]],
                                          dsems[u]).wait()

                pltpu.async_copy(ones, acc.at[idx_d.at[g]], dsems[u],
                                 add=True)
            return carry

        lax.fori_loop(0, _DEG_NIT // 2, body, 0)
        pltpu.make_async_copy(ones, acc.at[idx_d.at[0]], dsems[0]).wait()
        pltpu.make_async_copy(ones, acc.at[idx_d.at[0]], dsems[1]).wait()
        plsc.subcore_barrier()

        nvec = jnp.where(s == _NS - 1, _RLAST // 16, _RSZ // 16)

        @pl.when(s < _NS - 1)
        def _():
            pltpu.sync_copy(acc.at[pl.ds(r0, _RSZ)], dbuf.at[pl.ds(0, _RSZ)])

        @pl.when(s == _NS - 1)
        def _():
            pltpu.sync_copy(acc.at[pl.ds(r0, _RLAST)], dbuf.at[pl.ds(0, _RLAST)])

        def nbody(j, carry):
            v = dbuf[pl.ds(j * 16, 16)]
            dbuf[pl.ds(j * 16, 16)] = _rsqrt16(v + 1.0)
            return carry

        lax.fori_loop(0, nvec, nbody, 0)

        @pl.when(s < _NS - 1)
        def _():
            pltpu.sync_copy(dbuf.at[pl.ds(0, _RSZ)], out_hbm.at[pl.ds(r0, _RSZ)])

        @pl.when(s == _NS - 1)
        def _():
            pltpu.sync_copy(dbuf.at[pl.ds(0, _RLAST)], out_hbm.at[pl.ds(r0, _RLAST)])


@functools.partial(
    pl.kernel,
    mesh=_sc_mesh,
    out_type=jax.ShapeDtypeStruct((_NC, _N, _H), jnp.float32),
    scratch_types=[
        pltpu.VMEM((_NI, _CH), jnp.int32),
        pltpu.VMEM((_NI, _CH), jnp.int32),
        pltpu.VMEM((_TAIL,), jnp.int32),
        pltpu.VMEM((_TAIL,), jnp.int32),
        pltpu.VMEM((_NR, _CH, _H), jnp.float32),
        pltpu.VMEM_SHARED((_N, _H), jnp.float32),
        pltpu.SemaphoreType.DMA,
        pltpu.SemaphoreType.DMA,
        pltpu.SemaphoreType.DMA,
        pltpu.SemaphoreType.DMA,
        pltpu.SemaphoreType.DMA,
        pltpu.SemaphoreType.DMA,
        pltpu.SemaphoreType.DMA,
        pltpu.SemaphoreType.DMA,
        pltpu.SemaphoreType.DMA,
        pltpu.SemaphoreType.DMA,
        pltpu.SemaphoreType.DMA,
        pltpu.SemaphoreType.DMA,
    ],
)
def _edge_scatter(xws_hbm, src_hbm, dst_hbm, out_hbm,
                  idx_s, idx_d, tidx_s, tidx_d, rows, acc,
                  is0, is1, is2, is3, is4, is5, gs0, gs1, gs2,
                  ss0, ss1, ss2):
    isems = (is0, is1, is2, is3, is4, is5)
    gsems = (gs0, gs1, gs2)
    ssems = (ss0, ss1, ss2)
    c = lax.axis_index("c")
    s = lax.axis_index("s")
    w = c * _NS + s
    r0 = s * _RSZ
    base0 = w * _EPW

    def zr(i, carry):
        for j in range(_H // 16):
            rows[0, i, pl.ds(j * 16, 16)] = jnp.zeros((16,), jnp.float32)
        return carry

    lax.fori_loop(0, _CH, zr, 0)
    zch = rows.at[0]

    @pl.when(s < _NS - 1)
    def _():
        for k in range(_RSZ // _CH):
            pltpu.sync_copy(zch, acc.at[pl.ds(r0 + k * _CH, _CH)])

    @pl.when(s == _NS - 1)
    def _():
        for k in range(_RLAST // _CH):
            pltpu.sync_copy(zch, acc.at[pl.ds(r0 + k * _CH, _CH)])
        kz = _RLAST // _CH
        pltpu.sync_copy(rows.at[0, pl.ds(0, _RLAST - kz * _CH)],
                        acc.at[pl.ds(r0 + kz * _CH, _RLAST - kz * _CH)])

    plsc.subcore_barrier()

    def issue_idx(j, q6):
        pltpu.async_copy(src_hbm.at[pl.ds(base0 + j * _CH, _CH)],
                         idx_s.at[q6], isems[q6])
        pltpu.async_copy(dst_hbm.at[pl.ds(base0 + j * _CH, _CH)],
                         idx_d.at[q6], isems[q6])

    def wait_idx(q6):
        pltpu.make_async_copy(src_hbm.at[pl.ds(base0, _CH)], idx_s.at[q6],
                              isems[q6]).wait()
        pltpu.make_async_copy(dst_hbm.at[pl.ds(base0, _CH)], idx_d.at[q6],
                              isems[q6]).wait()

    def issue_gather(q6, q3):
        pltpu.async_copy(xws_hbm.at[idx_s.at[q6]], rows.at[q3], gsems[q3])

    def wait_gather(q3):
        pltpu.make_async_copy(xws_hbm.at[idx_s.at[0]], rows.at[q3],
                              gsems[q3]).wait()

    def issue_scat(q6, q3):
        pltpu.async_copy(rows.at[q3], acc.at[idx_d.at[q6]], ssems[q3],
                         add=True)

    def wait_scat(q3):
        pltpu.make_async_copy(rows.at[q3], acc.at[idx_d.at[0]],
                              ssems[q3]).wait()

    for j in range(4):
        issue_idx(j, j)
    wait_idx(0)
    issue_gather(0, 0)
    wait_idx(1)
    issue_gather(1, 1)

    def step(i, u):
        q3 = u % _NR
        q6 = u % _NI
        wait_gather(q3)
        issue_scat(q6, q3)
        j2 = i + 2
        r2 = (u + 2) % _NR
        q62 = (u + 2) % _NI

        @pl.when(j2 < _NITER)
        def _():
            @pl.when(j2 >= _NR)
            def _():
                wait_scat(r2)

            wait_idx(q62)
            issue_gather(q62, r2)

        j4 = i + 4
        q64 = (u + 4) % _NI

        @pl.when(j4 < _NITER)
        def _():
            issue_idx(j4, q64)

    def body(p, carry):
        for u in range(_NI):
            step(p * _NI + u, u)
        return carry

    lax.fori_loop(0, _NITER // _NI, body, 0)
    for q in range(_NR):
        wait_scat(q)

    tb = base0 + _NITER * _CH
    pltpu.sync_copy(src_hbm.at[pl.ds(tb, _TAIL)], tidx_s)
    pltpu.sync_copy(dst_hbm.at[pl.ds(tb, _TAIL)], tidx_d)
    trows = rows.at[0, pl.ds(0, _TAIL)]
    pltpu.async_copy(xws_hbm.at[tidx_s], trows, gs0).wait()
    pltpu.sync_copy(trows, acc.at[tidx_d], add=True)

    plsc.subcore_barrier()

    @pl.when(s < _NS - 1)
    def _():
        pltpu.sync_copy(acc.at[pl.ds(r0, _RSZ)], out_hbm.at[c, pl.ds(r0, _RSZ)])

    @pl.when(s == _NS - 1)
    def _():
        pltpu.sync_copy(acc.at[pl.ds(r0, _RLAST)],
                        out_hbm.at[c, pl.ds(r0, _RLAST)])


_RB = 1000
_G = _N // _RB


def _prep_body(dinv_ref, x_ref, w_ref, xw_ref, xws_ref):
    xw = jnp.dot(x_ref[...], w_ref[...], preferred_element_type=jnp.float32)
    xw_ref[...] = xw
    xws_ref[...] = dinv_ref[...] * xw


_prep = pl.pallas_call(
    _prep_body,
    grid=(_G,),
    in_specs=[
        pl.BlockSpec((_RB, 1), lambda i: (i, 0)),
        pl.BlockSpec((_RB, _D), lambda i: (i, 0)),
        pl.BlockSpec((_D, _H), lambda i: (0, 0)),
    ],
    out_specs=[
        pl.BlockSpec((_RB, _H), lambda i: (i, 0)),
        pl.BlockSpec((_RB, _H), lambda i: (i, 0)),
    ],
    out_shape=[
        jax.ShapeDtypeStruct((_N, _H), jnp.float32),
        jax.ShapeDtypeStruct((_N, _H), jnp.float32),
    ],
)


def _agg_body(scat_ref, xw_ref, dinv_ref, b_ref, hpre_ref, st_ref):
    dinv = dinv_ref[...]
    agg = (dinv * (scat_ref[0] + scat_ref[1])
           + (dinv * dinv) * xw_ref[...] + b_ref[...])
    h = jnp.maximum(agg, 0.0)
    hpre_ref[...] = h
    st_ref[0, 0, :] = jnp.sum(h, axis=0)
    st_ref[0, 1, :] = jnp.sum(h * h, axis=0)


_agg = pl.pallas_call(
    _agg_body,
    grid=(_G,),
    in_specs=[
        pl.BlockSpec((_NC, _RB, _H), lambda i: (0, i, 0)),
        pl.BlockSpec((_RB, _H), lambda i: (i, 0)),
        pl.BlockSpec((_RB, 1), lambda i: (i, 0)),
        pl.BlockSpec((1, _H), lambda i: (0, 0)),
    ],
    out_specs=[
        pl.BlockSpec((_RB, _H), lambda i: (i, 0)),
        pl.BlockSpec((1, 2, _H), lambda i: (i, 0, 0)),
    ],
    out_shape=[
        jax.ShapeDtypeStruct((_N, _H), jnp.float32),
        jax.ShapeDtypeStruct((_G, 2, _H), jnp.float32),
    ],
)


def _bn_stats(st):
    mu = jnp.sum(st[:, 0, :], axis=0) * (1.0 / _N)
    ex2 = jnp.sum(st[:, 1, :], axis=0) * (1.0 / _N)
    var = ex2 - mu * mu
    rs = lax.rsqrt(var + _EPS)
    return mu, rs


def _bn_mm_body(hpre_ref, st_ref, g_ref, be_ref, w2_ref, dinv_ref,
                h_ref, xw2_ref, xws2_ref):
    mu, rs = _bn_stats(st_ref[...])
    h = (hpre_ref[...] - mu) * (rs * g_ref[...]) + be_ref[...]
    h_ref[...] = h
    xw2 = jnp.dot(h, w2_ref[...], preferred_element_type=jnp.float32)
    xw2_ref[...] = xw2
    xws2_ref[...] = dinv_ref[...] * xw2


_bn_mm = pl.pallas_call(
    _bn_mm_body,
    grid=(_G,),
    in_specs=[
        pl.BlockSpec((_RB, _H), lambda i: (i, 0)),
        pl.BlockSpec((_G, 2, _H), lambda i: (0, 0, 0)),
        pl.BlockSpec((1, _H), lambda i: (0, 0)),
        pl.BlockSpec((1, _H), lambda i: (0, 0)),
        pl.BlockSpec((_D, _H), lambda i: (0, 0)),
        pl.BlockSpec((_RB, 1), lambda i: (i, 0)),
    ],
    out_specs=[
        pl.BlockSpec((_RB, _H), lambda i: (i, 0)),
        pl.BlockSpec((_RB, _H), lambda i: (i, 0)),
        pl.BlockSpec((_RB, _H), lambda i: (i, 0)),
    ],
    out_shape=[
        jax.ShapeDtypeStruct((_N, _H), jnp.float32),
        jax.ShapeDtypeStruct((_N, _H), jnp.float32),
        jax.ShapeDtypeStruct((_N, _H), jnp.float32),
    ],
)


def _bn_body(hpre_ref, st_ref, g_ref, be_ref, h_ref):
    mu, rs = _bn_stats(st_ref[...])
    h_ref[...] = (hpre_ref[...] - mu) * (rs * g_ref[...]) + be_ref[...]


_bn = pl.pallas_call(
    _bn_body,
    grid=(_G,),
    in_specs=[
        pl.BlockSpec((_RB, _H), lambda i: (i, 0)),
        pl.BlockSpec((_G, 2, _H), lambda i: (0, 0, 0)),
        pl.BlockSpec((1, _H), lambda i: (0, 0)),
        pl.BlockSpec((1, _H), lambda i: (0, 0)),
    ],
    out_specs=pl.BlockSpec((_RB, _H), lambda i: (i, 0)),
    out_shape=jax.ShapeDtypeStruct((_N, _H), jnp.float32),
)


def kernel(node_features, edge_indices, W1, b1, g1, be1, W2, b2, g2, be2):
    src = edge_indices[0]
    dst = edge_indices[1]
    dst_deg = dst.reshape(_NS, _DEG_NIT, _DEG_CH2)
    dinv = _deg_dinv(dst_deg)
    dinv_col = dinv.reshape(_N, 1)

    xw1, xws1 = _prep(dinv_col, node_features, W1)
    scat1 = _edge_scatter(xws1, src, dst)
    hpre1, st1 = _agg(scat1, xw1, dinv_col, b1.reshape(1, _H))
    h1, xw2, xws2 = _bn_mm(hpre1, st1, g1.reshape(1, _H), be1.reshape(1, _H),
                           W2, dinv_col)

    scat2 = _edge_scatter(xws2, src, dst)
    hpre2, st2 = _agg(scat2, xw2, dinv_col, b2.reshape(1, _H))
    h2 = _bn(hpre2, st2, g2.reshape(1, _H), be2.reshape(1, _H))
    return (h1, h2)


# speedup vs baseline: 1.0518x; 1.0135x over previous
import functools

import jax
import jax.numpy as jnp
from jax import lax
from jax.experimental import pallas as pl
from jax.experimental.pallas import tpu as pltpu
from jax.experimental.pallas import tpu_sc as plsc

_N = 10000
_E = 320000
_D = 128
_H = 128
_EPS = 1e-5

_NC = 2
_NS = 16
_NW = _NC * _NS
_EPW = _E // _NW
_CH = 64
_NITER = _EPW // _CH
_TAIL = _EPW - _NITER * _CH
_NR = 3
_NI = 6

_RSZ = 640
_RLAST = _N - (_NS - 1) * _RSZ

_sc_mesh = plsc.VectorSubcoreMesh(core_axis_name="c", subcore_axis_name="s")


def _rsqrt16(x):
    xi = lax.bitcast_convert_type(x, jnp.int32)
    yi = jnp.int32(0x5F3759DF) - lax.shift_right_arithmetic(xi, 1)
    y = lax.bitcast_convert_type(yi, jnp.float32)
    for _ in range(3):
        y = y * (1.5 - 0.5 * x * y * y)
    return y


_DEG_CH2 = 80
_DEG_EPT = _E // _NS
_DEG_NIT = _DEG_EPT // _DEG_CH2


@functools.partial(
    pl.kernel,
    mesh=_sc_mesh,
    out_type=jax.ShapeDtypeStruct((_N,), jnp.float32),
    scratch_types=[
        pltpu.VMEM((_DEG_NIT, _DEG_CH2), jnp.int32),
        pltpu.VMEM((_DEG_CH2,), jnp.float32),
        pltpu.VMEM((_RSZ,), jnp.float32),
        pltpu.VMEM_SHARED((_N,), jnp.float32),
        pltpu.SemaphoreType.DMA,
        pltpu.SemaphoreType.DMA,
    ],
)
def _deg_dinv(dstd_hbm, out_hbm, idx_d, ones, dbuf, acc, dg0, dg1):
    c = lax.axis_index("c")
    s = lax.axis_index("s")

    @pl.when(c == 0)
    def _core0():
        for j in range(_DEG_CH2 // 16):
            ones[pl.ds(j * 16, 16)] = jnp.ones((16,), jnp.float32)

        def zb(j, carry):
            dbuf[pl.ds(j * 16, 16)] = jnp.zeros((16,), jnp.float32)
            return carry

        lax.fori_loop(0, _RSZ // 16, zb, 0)
        r0 = s * _RSZ

        @pl.when(s < _NS - 1)
        def _():
            pltpu.sync_copy(dbuf.at[pl.ds(0, _RSZ)], acc.at[pl.ds(r0, _RSZ)])

        @pl.when(s == _NS - 1)
        def _():
            pltpu.sync_copy(dbuf.at[pl.ds(0, _RLAST)], acc.at[pl.ds(r0, _RLAST)])

        pltpu.sync_copy(dstd_hbm.at[s], idx_d)
        plsc.subcore_barrier()

        dsems = (dg0, dg1)

        def body(p, carry):
            for u in range(2):
                g = p * 2 + u

                @pl.when(g >= 2)
                def _():
                    pltpu.make_async_copy(ones, acc.at[idx_d.at[0]],
                                          dsems[u]).wait()

                pltpu.async_copy(ones, acc.at[idx_d.at[g]], dsems[u],
                                 add=True)
            return carry

        lax.fori_loop(0, _DEG_NIT // 2, body, 0)
        pltpu.make_async_copy(ones, acc.at[idx_d.at[0]], dsems[0]).wait()
        pltpu.make_async_copy(ones, acc.at[idx_d.at[0]], dsems[1]).wait()
        plsc.subcore_barrier()

        nvec = jnp.where(s == _NS - 1, _RLAST // 16, _RSZ // 16)

        @pl.when(s < _NS - 1)
        def _():
            pltpu.sync_copy(acc.at[pl.ds(r0, _RSZ)], dbuf.at[pl.ds(0, _RSZ)])

        @pl.when(s == _NS - 1)
        def _():
            pltpu.sync_copy(acc.at[pl.ds(r0, _RLAST)], dbuf.at[pl.ds(0, _RLAST)])

        def nbody(j, carry):
            v = dbuf[pl.ds(j * 16, 16)]
            dbuf[pl.ds(j * 16, 16)] = _rsqrt16(v + 1.0)
            return carry

        lax.fori_loop(0, nvec, nbody, 0)

        @pl.when(s < _NS - 1)
        def _():
            pltpu.sync_copy(dbuf.at[pl.ds(0, _RSZ)], out_hbm.at[pl.ds(r0, _RSZ)])

        @pl.when(s == _NS - 1)
        def _():
            pltpu.sync_copy(dbuf.at[pl.ds(0, _RLAST)], out_hbm.at[pl.ds(r0, _RLAST)])


@functools.partial(
    pl.kernel,
    mesh=_sc_mesh,
    out_type=jax.ShapeDtypeStruct((_NC, _N, _H), jnp.float32),
    scratch_types=[
        pltpu.VMEM((_NI, _CH), jnp.int32),
        pltpu.VMEM((_NI, _CH), jnp.int32),
        pltpu.VMEM((_TAIL,), jnp.int32),
        pltpu.VMEM((_TAIL,), jnp.int32),
        pltpu.VMEM((_NR, _CH, _H), jnp.float32),
        pltpu.VMEM_SHARED((_N, _H), jnp.float32),
        pltpu.SemaphoreType.DMA,
        pltpu.SemaphoreType.DMA,
        pltpu.SemaphoreType.DMA,
        pltpu.SemaphoreType.DMA,
        pltpu.SemaphoreType.DMA,
        pltpu.SemaphoreType.DMA,
        pltpu.SemaphoreType.DMA,
        pltpu.SemaphoreType.DMA,
        pltpu.SemaphoreType.DMA,
        pltpu.SemaphoreType.DMA,
        pltpu.SemaphoreType.DMA,
        pltpu.SemaphoreType.DMA,
    ],
)
def _edge_scatter(xws_hbm, src_hbm, dst_hbm, out_hbm,
                  idx_s, idx_d, tidx_s, tidx_d, rows, acc,
                  is0, is1, is2, is3, is4, is5, gs0, gs1, gs2,
                  ss0, ss1, ss2):
    isems = (is0, is1, is2, is3, is4, is5)
    gsems = (gs0, gs1, gs2)
    ssems = (ss0, ss1, ss2)
    c = lax.axis_index("c")
    s = lax.axis_index("s")
    w = c * _NS + s
    r0 = s * _RSZ
    base0 = w * _EPW

    def zr(i, carry):
        for j in range(_H // 16):
            rows[0, i, pl.ds(j * 16, 16)] = jnp.zeros((16,), jnp.float32)
        return carry

    lax.fori_loop(0, _CH, zr, 0)
    zch = rows.at[0]

    @pl.when(s < _NS - 1)
    def _():
        for k in range(_RSZ // _CH):
            pltpu.sync_copy(zch, acc.at[pl.ds(r0 + k * _CH, _CH)])

    @pl.when(s == _NS - 1)
    def _():
        for k in range(_RLAST // _CH):
            pltpu.sync_copy(zch, acc.at[pl.ds(r0 + k * _CH, _CH)])
        kz = _RLAST // _CH
        pltpu.sync_copy(rows.at[0, pl.ds(0, _RLAST - kz * _CH)],
                        acc.at[pl.ds(r0 + kz * _CH, _RLAST - kz * _CH)])

    plsc.subcore_barrier()

    def issue_idx(j, q6):
        pltpu.async_copy(src_hbm.at[pl.ds(base0 + j * _CH, _CH)],
                         idx_s.at[q6], isems[q6])
        pltpu.async_copy(dst_hbm.at[pl.ds(base0 + j * _CH, _CH)],
                         idx_d.at[q6], isems[q6])

    def wait_idx(q6):
        pltpu.make_async_copy(src_hbm.at[pl.ds(base0, _CH)], idx_s.at[q6],
                              isems[q6]).wait()
        pltpu.make_async_copy(dst_hbm.at[pl.ds(base0, _CH)], idx_d.at[q6],
                              isems[q6]).wait()

    def issue_gather(q6, q3):
        pltpu.async_copy(xws_hbm.at[idx_s.at[q6]], rows.at[q3], gsems[q3])

    def wait_gather(q3):
        pltpu.make_async_copy(xws_hbm.at[idx_s.at[0]], rows.at[q3],
                              gsems[q3]).wait()

    def issue_scat(q6, q3):
        pltpu.async_copy(rows.at[q3], acc.at[idx_d.at[q6]], ssems[q3],
                         add=True)

    def wait_scat(q3):
        pltpu.make_async_copy(rows.at[q3], acc.at[idx_d.at[0]],
                              ssems[q3]).wait()

    for j in range(4):
        issue_idx(j, j)
    wait_idx(0)
    issue_gather(0, 0)
    wait_idx(1)
    issue_gather(1, 1)

    def step(i, u):
        q3 = u % _NR
        q6 = u % _NI
        wait_gather(q3)
        issue_scat(q6, q3)
        j2 = i + 2
        r2 = (u + 2) % _NR
        q62 = (u + 2) % _NI

        @pl.when(j2 < _NITER)
        def _():
            @pl.when(j2 >= _NR)
            def _():
                wait_scat(r2)

            wait_idx(q62)
            issue_gather(q62, r2)

        j4 = i + 4
        q64 = (u + 4) % _NI

        @pl.when(j4 < _NITER)
        def _():
            issue_idx(j4, q64)

    def body(p, carry):
        for u in range(_NI):
            step(p * _NI + u, u)
        return carry

    lax.fori_loop(0, _NITER // _NI, body, 0)
    for q in range(_NR):
        wait_scat(q)

    tb = base0 + _NITER * _CH
    pltpu.sync_copy(src_hbm.at[pl.ds(tb, _TAIL)], tidx_s)
    pltpu.sync_copy(dst_hbm.at[pl.ds(tb, _TAIL)], tidx_d)
    trows = rows.at[0, pl.ds(0, _TAIL)]
    pltpu.async_copy(xws_hbm.at[tidx_s], trows, gs0).wait()
    pltpu.sync_copy(trows, acc.at[tidx_d], add=True)

    plsc.subcore_barrier()

    @pl.when(s < _NS - 1)
    def _():
        pltpu.sync_copy(acc.at[pl.ds(r0, _RSZ)], out_hbm.at[c, pl.ds(r0, _RSZ)])

    @pl.when(s == _NS - 1)
    def _():
        pltpu.sync_copy(acc.at[pl.ds(r0, _RLAST)],
                        out_hbm.at[c, pl.ds(r0, _RLAST)])


_RB = 1000
_G = _N // _RB


def _mm_body(x_ref, w_ref, xw_ref):
    xw_ref[...] = jnp.dot(x_ref[...], w_ref[...],
                          preferred_element_type=jnp.float32)


_mm = pl.pallas_call(
    _mm_body,
    grid=(_G,),
    in_specs=[
        pl.BlockSpec((_RB, _D), lambda i: (i, 0)),
        pl.BlockSpec((_D, _H), lambda i: (0, 0)),
    ],
    out_specs=pl.BlockSpec((_RB, _H), lambda i: (i, 0)),
    out_shape=jax.ShapeDtypeStruct((_N, _H), jnp.float32),
)


def _scale_body(dinv_ref, xw_ref, xws_ref):
    xws_ref[...] = dinv_ref[...] * xw_ref[...]


_scale = pl.pallas_call(
    _scale_body,
    grid=(_G,),
    in_specs=[
        pl.BlockSpec((_RB, 1), lambda i: (i, 0)),
        pl.BlockSpec((_RB, _H), lambda i: (i, 0)),
    ],
    out_specs=pl.BlockSpec((_RB, _H), lambda i: (i, 0)),
    out_shape=jax.ShapeDtypeStruct((_N, _H), jnp.float32),
)


def _agg_body(scat_ref, xw_ref, dinv_ref, b_ref, hpre_ref, st_ref):
    dinv = dinv_ref[...]
    agg = (dinv * (scat_ref[0] + scat_ref[1])
           + (dinv * dinv) * xw_ref[...] + b_ref[...])
    h = jnp.maximum(agg, 0.0)
    hpre_ref[...] = h
    st_ref[0, 0, :] = jnp.sum(h, axis=0)
    st_ref[0, 1, :] = jnp.sum(h * h, axis=0)


_agg = pl.pallas_call(
    _agg_body,
    grid=(_G,),
    in_specs=[
        pl.BlockSpec((_NC, _RB, _H), lambda i: (0, i, 0)),
        pl.BlockSpec((_RB, _H), lambda i: (i, 0)),
        pl.BlockSpec((_RB, 1), lambda i: (i, 0)),
        pl.BlockSpec((1, _H), lambda i: (0, 0)),
    ],
    out_specs=[
        pl.BlockSpec((_RB, _H), lambda i: (i, 0)),
        pl.BlockSpec((1, 2, _H), lambda i: (i, 0, 0)),
    ],
    out_shape=[
        jax.ShapeDtypeStruct((_N, _H), jnp.float32),
        jax.ShapeDtypeStruct((_G, 2, _H), jnp.float32),
    ],
)


def _bn_stats(st):
    mu = jnp.sum(st[:, 0, :], axis=0) * (1.0 / _N)
    ex2 = jnp.sum(st[:, 1, :], axis=0) * (1.0 / _N)
    var = ex2 - mu * mu
    rs = lax.rsqrt(var + _EPS)
    return mu, rs


def _bn_mm_body(hpre_ref, st_ref, g_ref, be_ref, w2_ref, dinv_ref,
                h_ref, xw2_ref, xws2_ref):
    mu, rs = _bn_stats(st_ref[...])
    h = (hpre_ref[...] - mu) * (rs * g_ref[...]) + be_ref[...]
    h_ref[...] = h
    xw2 = jnp.dot(h, w2_ref[...], preferred_element_type=jnp.float32)
    xw2_ref[...] = xw2
    xws2_ref[...] = dinv_ref[...] * xw2


_bn_mm = pl.pallas_call(
    _bn_mm_body,
    grid=(_G,),
    in_specs=[
        pl.BlockSpec((_RB, _H), lambda i: (i, 0)),
        pl.BlockSpec((_G, 2, _H), lambda i: (0, 0, 0)),
        pl.BlockSpec((1, _H), lambda i: (0, 0)),
        pl.BlockSpec((1, _H), lambda i: (0, 0)),
        pl.BlockSpec((_D, _H), lambda i: (0, 0)),
        pl.BlockSpec((_RB, 1), lambda i: (i, 0)),
    ],
    out_specs=[
        pl.BlockSpec((_RB, _H), lambda i: (i, 0)),
        pl.BlockSpec((_RB, _H), lambda i: (i, 0)),
        pl.BlockSpec((_RB, _H), lambda i: (i, 0)),
    ],
    out_shape=[
        jax.ShapeDtypeStruct((_N, _H), jnp.float32),
        jax.ShapeDtypeStruct((_N, _H), jnp.float32),
        jax.ShapeDtypeStruct((_N, _H), jnp.float32),
    ],
)


def _bn_body(hpre_ref, st_ref, g_ref, be_ref, h_ref):
    mu, rs = _bn_stats(st_ref[...])
    h_ref[...] = (hpre_ref[...] - mu) * (rs * g_ref[...]) + be_ref[...]


_bn = pl.pallas_call(
    _bn_body,
    grid=(_G,),
    in_specs=[
        pl.BlockSpec((_RB, _H), lambda i: (i, 0)),
        pl.BlockSpec((_G, 2, _H), lambda i: (0, 0, 0)),
        pl.BlockSpec((1, _H), lambda i: (0, 0)),
        pl.BlockSpec((1, _H), lambda i: (0, 0)),
    ],
    out_specs=pl.BlockSpec((_RB, _H), lambda i: (i, 0)),
    out_shape=jax.ShapeDtypeStruct((_N, _H), jnp.float32),
)


def kernel(node_features, edge_indices, W1, b1, g1, be1, W2, b2, g2, be2):
    src = edge_indices[0]
    dst = edge_indices[1]
    dst_deg = dst.reshape(_NS, _DEG_NIT, _DEG_CH2)
    xw1 = _mm(node_features, W1)
    dinv = _deg_dinv(dst_deg)
    dinv_col = dinv.reshape(_N, 1)
    xws1 = _scale(dinv_col, xw1)
    scat1 = _edge_scatter(xws1, src, dst)
    hpre1, st1 = _agg(scat1, xw1, dinv_col, b1.reshape(1, _H))
    h1, xw2, xws2 = _bn_mm(hpre1, st1, g1.reshape(1, _H), be1.reshape(1, _H),
                           W2, dinv_col)

    scat2 = _edge_scatter(xws2, src, dst)
    hpre2, st2 = _agg(scat2, xw2, dinv_col, b2.reshape(1, _H))
    h2 = _bn(hpre2, st2, g2.reshape(1, _H), be2.reshape(1, _H))
    return (h1, h2)
